# initial kernel scaffold (unmeasured)
import jax
import jax.numpy as jnp
from jax import lax
from jax.experimental import pallas as pl
from jax.experimental.pallas import tpu as pltpu

N_DEV = 16
M, K_TOT, N = 4096, 4096, 8192
K_SH = K_TOT // N_DEV
TM = 512


def kernel(x, w_mat, scale_x, scale_w):
    x8 = x.astype(jnp.float8_e4m3fn)
    w8 = w_mat.astype(jnp.float8_e4m3fn)

    def body(x_ref, w_ref, sx_ref, sw_ref, out_ref,
             xg, wg, acc, lx_sem, lw_sem, out_sem,
             xs_send, xs_recv, ws_send, ws_recv):
        my = lax.axis_index("i")
        right = lax.rem(my + 1, N_DEV)
        left = lax.rem(my + N_DEV - 1, N_DEV)

        barrier = pltpu.get_barrier_semaphore()
        pl.semaphore_signal(barrier, inc=1, device_id=(left,),
                            device_id_type=pl.DeviceIdType.MESH)
        pl.semaphore_signal(barrier, inc=1, device_id=(right,),
                            device_id_type=pl.DeviceIdType.MESH)
        pl.semaphore_wait(barrier, 2)

        cx = pltpu.make_async_copy(x_ref, xg.at[my], lx_sem)
        cw = pltpu.make_async_copy(w_ref, wg.at[my], lw_sem)
        cx.start()
        cw.start()
        cx.wait()
        cw.wait()

        for h in range(N_DEV - 1):
            s = lax.rem(my - h + N_DEV, N_DEV)
            rx = pltpu.make_async_remote_copy(
                src_ref=xg.at[s], dst_ref=xg.at[s],
                send_sem=xs_send.at[h], recv_sem=xs_recv.at[h],
                device_id=(right,), device_id_type=pl.DeviceIdType.MESH)
            rw = pltpu.make_async_remote_copy(
                src_ref=wg.at[s], dst_ref=wg.at[s],
                send_sem=ws_send.at[h], recv_sem=ws_recv.at[h],
                device_id=(right,), device_id_type=pl.DeviceIdType.MESH)
            rx.start()
            rw.start()
            rx.wait()
            rw.wait()

        alpha = sx_ref[0] * sw_ref[0]
        for mi in range(M // TM):
            m0 = mi * TM
            acc[...] = jnp.dot(xg[0, m0:m0 + TM, :], wg[0],
                               preferred_element_type=jnp.float32)
            for k in range(1, N_DEV):
                acc[...] += jnp.dot(xg[k, m0:m0 + TM, :], wg[k],
                                    preferred_element_type=jnp.float32)
            y = acc[...] * alpha
            yc = jnp.clip(y, -60.0, 60.0)
            acc[...] = y * (1.0 / (1.0 + jnp.exp(-yc)))
            cp = pltpu.make_async_copy(
                acc, out_ref.at[pl.ds(m0, TM), :], out_sem)
            cp.start()
            cp.wait()

    return pl.pallas_call(
        body,
        out_shape=jax.ShapeDtypeStruct((M, N), jnp.float32),
        in_specs=[
            pl.BlockSpec(memory_space=pltpu.VMEM),
            pl.BlockSpec(memory_space=pltpu.VMEM),
            pl.BlockSpec(memory_space=pltpu.SMEM),
            pl.BlockSpec(memory_space=pltpu.SMEM),
        ],
        out_specs=pl.BlockSpec(memory_space=pltpu.ANY),
        scratch_shapes=[
            pltpu.VMEM((N_DEV, M, K_SH), jnp.float8_e4m3fn),
            pltpu.VMEM((N_DEV, K_SH, N), jnp.float8_e4m3fn),
            pltpu.VMEM((TM, N), jnp.float32),
            pltpu.SemaphoreType.DMA,
            pltpu.SemaphoreType.DMA,
            pltpu.SemaphoreType.DMA,
            pltpu.SemaphoreType.DMA((N_DEV - 1,)),
            pltpu.SemaphoreType.DMA((N_DEV - 1,)),
            pltpu.SemaphoreType.DMA((N_DEV - 1,)),
            pltpu.SemaphoreType.DMA((N_DEV - 1,)),
        ],
        compiler_params=pltpu.CompilerParams(collective_id=0),
    )(x8, w8, scale_x, scale_w)


# baseline (device time: 841014 ns/iter reference)
import jax
import jax.numpy as jnp
from jax import lax
from jax.experimental import pallas as pl
from jax.experimental.pallas import tpu as pltpu

N_DEV = 16
M, K_TOT, N = 4096, 4096, 8192
K_SH = K_TOT // N_DEV
TM = 512
TN = 1024


def kernel(x, w_mat, scale_x, scale_w):
    x8 = x.astype(jnp.float8_e4m3fn)
    w8 = w_mat.astype(jnp.float8_e4m3fn)

    def body(x_ref, w_ref, sx_ref, sw_ref, o_ref, wf_ref,
             xg, wstage, lx_sem, lw_sem, ls_sem,
             xs_send, xs_recv, ws_send, ws_recv):
        ni = pl.program_id(0)
        mi = pl.program_id(1)
        pid = ni * (M // TM) + mi
        my = lax.axis_index("i")
        right = lax.rem(my + 1, N_DEV)
        left = lax.rem(my + N_DEV - 1, N_DEV)

        @pl.when(pid == 0)
        def _gather():
            barrier = pltpu.get_barrier_semaphore()
            pl.semaphore_signal(barrier, inc=1, device_id=(left,),
                                device_id_type=pl.DeviceIdType.MESH)
            pl.semaphore_signal(barrier, inc=1, device_id=(right,),
                                device_id_type=pl.DeviceIdType.MESH)
            pl.semaphore_wait(barrier, 2)

            cx = pltpu.make_async_copy(
                x_ref, xg.at[:, pl.ds(my * K_SH, K_SH)], lx_sem)
            cw = pltpu.make_async_copy(
                w_ref, wf_ref.at[pl.ds(my * K_SH, K_SH), :], lw_sem)
            cx.start()
            cw.start()
            cx.wait()
            cw.wait()

            for h in range(N_DEV - 1):
                s = lax.rem(my - h + N_DEV, N_DEV)
                xsl = xg.at[:, pl.ds(s * K_SH, K_SH)]
                wsl = wf_ref.at[pl.ds(s * K_SH, K_SH), :]
                rx = pltpu.make_async_remote_copy(
                    src_ref=xsl, dst_ref=xsl,
                    send_sem=xs_send.at[h], recv_sem=xs_recv.at[h],
                    device_id=(right,), device_id_type=pl.DeviceIdType.MESH)
                rw = pltpu.make_async_remote_copy(
                    src_ref=wsl, dst_ref=wsl,
                    send_sem=ws_send.at[h], recv_sem=ws_recv.at[h],
                    device_id=(right,), device_id_type=pl.DeviceIdType.MESH)
                rx.start()
                rw.start()
                rx.wait()
                rw.wait()

        @pl.when(mi == 0)
        def _stage_w():
            cs = pltpu.make_async_copy(
                wf_ref.at[:, pl.ds(ni * TN, TN)], wstage, ls_sem)
            cs.start()
            cs.wait()

        m0 = mi * TM
        acc = jnp.dot(xg[pl.ds(m0, TM), :], wstage[...],
                      preferred_element_type=jnp.float32)
        y = acc * (sx_ref[0] * sw_ref[0])
        yc = jnp.clip(y, -60.0, 60.0)
        o_ref[...] = y * (1.0 / (1.0 + jnp.exp(-yc)))

    out, _ = pl.pallas_call(
        body,
        grid=(N // TN, M // TM),
        out_shape=[
            jax.ShapeDtypeStruct((M, N), jnp.float32),
            jax.ShapeDtypeStruct((K_TOT, N), jnp.float8_e4m3fn),
        ],
        in_specs=[
            pl.BlockSpec(memory_space=pltpu.VMEM),
            pl.BlockSpec(memory_space=pltpu.VMEM),
            pl.BlockSpec(memory_space=pltpu.SMEM),
            pl.BlockSpec(memory_space=pltpu.SMEM),
        ],
        out_specs=[
            pl.BlockSpec((TM, TN), lambda n, m: (m, n)),
            pl.BlockSpec(memory_space=pl.ANY),
        ],
        scratch_shapes=[
            pltpu.VMEM((M, K_TOT), jnp.float8_e4m3fn),
            pltpu.VMEM((K_TOT, TN), jnp.float8_e4m3fn),
            pltpu.SemaphoreType.DMA,
            pltpu.SemaphoreType.DMA,
            pltpu.SemaphoreType.DMA,
            pltpu.SemaphoreType.DMA((N_DEV - 1,)),
            pltpu.SemaphoreType.DMA((N_DEV - 1,)),
            pltpu.SemaphoreType.DMA((N_DEV - 1,)),
            pltpu.SemaphoreType.DMA((N_DEV - 1,)),
        ],
        compiler_params=pltpu.CompilerParams(
            collective_id=0,
            dimension_semantics=("arbitrary", "arbitrary"),
        ),
    )(x8, w8, scale_x, scale_w)
    return out


# device time: 604049 ns/iter; 1.3923x vs baseline; 1.3923x over previous
import jax
import jax.numpy as jnp
from jax import lax
from jax.experimental import pallas as pl
from jax.experimental.pallas import tpu as pltpu

N_DEV = 16
M, K_TOT, N = 4096, 4096, 8192
K_SH = K_TOT // N_DEV
TM = 512
TN = 1024


def kernel(x, w_mat, scale_x, scale_w):
    x8 = x.astype(jnp.float8_e4m3fn)
    w8 = w_mat.astype(jnp.float8_e4m3fn)

    def body(x_ref, w_ref, sx_ref, sw_ref, o_ref, wf_ref,
             xg, wstage, lx_sem, lw_sem, ls_sem,
             xs_send, xs_recv, ws_send, ws_recv,
             xl_send, xl_recv, wl_send, wl_recv):
        ni = pl.program_id(0)
        mi = pl.program_id(1)
        pid = ni * (M // TM) + mi
        my = lax.axis_index("i")
        right = lax.rem(my + 1, N_DEV)
        left = lax.rem(my + N_DEV - 1, N_DEV)

        @pl.when(pid == 0)
        def _gather():
            barrier = pltpu.get_barrier_semaphore()
            pl.semaphore_signal(barrier, inc=1, device_id=(left,),
                                device_id_type=pl.DeviceIdType.MESH)
            pl.semaphore_signal(barrier, inc=1, device_id=(right,),
                                device_id_type=pl.DeviceIdType.MESH)
            pl.semaphore_wait(barrier, 2)

            cx = pltpu.make_async_copy(
                x_ref, xg.at[:, pl.ds(my * K_SH, K_SH)], lx_sem)
            cw = pltpu.make_async_copy(
                w_ref, wf_ref.at[pl.ds(my * K_SH, K_SH), :], lw_sem)
            cx.start()
            cw.start()
            cx.wait()
            cw.wait()

            H_R, H_L = 8, 7
            for h in range(H_R):
                sr = lax.rem(my - h + N_DEV, N_DEV)
                xr = xg.at[:, pl.ds(sr * K_SH, K_SH)]
                wr = wf_ref.at[pl.ds(sr * K_SH, K_SH), :]
                rxr = pltpu.make_async_remote_copy(
                    src_ref=xr, dst_ref=xr,
                    send_sem=xs_send.at[h], recv_sem=xs_recv.at[h],
                    device_id=(right,), device_id_type=pl.DeviceIdType.MESH)
                rwr = pltpu.make_async_remote_copy(
                    src_ref=wr, dst_ref=wr,
                    send_sem=ws_send.at[h], recv_sem=ws_recv.at[h],
                    device_id=(right,), device_id_type=pl.DeviceIdType.MESH)
                rxr.start()
                rwr.start()
                if h < H_L:
                    sl = lax.rem(my + h, N_DEV)
                    xl = xg.at[:, pl.ds(sl * K_SH, K_SH)]
                    wl = wf_ref.at[pl.ds(sl * K_SH, K_SH), :]
                    rxl = pltpu.make_async_remote_copy(
                        src_ref=xl, dst_ref=xl,
                        send_sem=xl_send.at[h], recv_sem=xl_recv.at[h],
                        device_id=(left,),
                        device_id_type=pl.DeviceIdType.MESH)
                    rwl = pltpu.make_async_remote_copy(
                        src_ref=wl, dst_ref=wl,
                        send_sem=wl_send.at[h], recv_sem=wl_recv.at[h],
                        device_id=(left,),
                        device_id_type=pl.DeviceIdType.MESH)
                    rxl.start()
                    rwl.start()
                    rxl.wait()
                    rwl.wait()
                rxr.wait()
                rwr.wait()

        @pl.when(mi == 0)
        def _stage_w():
            cs = pltpu.make_async_copy(
                wf_ref.at[:, pl.ds(ni * TN, TN)], wstage, ls_sem)
            cs.start()
            cs.wait()

        m0 = mi * TM
        acc = jnp.dot(xg[pl.ds(m0, TM), :], wstage[...],
                      preferred_element_type=jnp.float32)
        y = acc * (sx_ref[0] * sw_ref[0])
        yc = jnp.clip(y, -60.0, 60.0)
        o_ref[...] = y * (1.0 / (1.0 + jnp.exp(-yc)))

    out, _ = pl.pallas_call(
        body,
        grid=(N // TN, M // TM),
        out_shape=[
            jax.ShapeDtypeStruct((M, N), jnp.float32),
            jax.ShapeDtypeStruct((K_TOT, N), jnp.float8_e4m3fn),
        ],
        in_specs=[
            pl.BlockSpec(memory_space=pltpu.VMEM),
            pl.BlockSpec(memory_space=pltpu.VMEM),
            pl.BlockSpec(memory_space=pltpu.SMEM),
            pl.BlockSpec(memory_space=pltpu.SMEM),
        ],
        out_specs=[
            pl.BlockSpec((TM, TN), lambda n, m: (m, n)),
            pl.BlockSpec(memory_space=pl.ANY),
        ],
        scratch_shapes=[
            pltpu.VMEM((M, K_TOT), jnp.float8_e4m3fn),
            pltpu.VMEM((K_TOT, TN), jnp.float8_e4m3fn),
            pltpu.SemaphoreType.DMA,
            pltpu.SemaphoreType.DMA,
            pltpu.SemaphoreType.DMA,
            pltpu.SemaphoreType.DMA((8,)),
            pltpu.SemaphoreType.DMA((8,)),
            pltpu.SemaphoreType.DMA((8,)),
            pltpu.SemaphoreType.DMA((8,)),
            pltpu.SemaphoreType.DMA((7,)),
            pltpu.SemaphoreType.DMA((7,)),
            pltpu.SemaphoreType.DMA((7,)),
            pltpu.SemaphoreType.DMA((7,)),
        ],
        compiler_params=pltpu.CompilerParams(
            collective_id=0,
            dimension_semantics=("arbitrary", "arbitrary"),
        ),
    )(x8, w8, scale_x, scale_w)
    return out


# device time: 580222 ns/iter; 1.4495x vs baseline; 1.0411x over previous
import jax
import jax.numpy as jnp
from jax import lax
from jax.experimental import pallas as pl
from jax.experimental.pallas import tpu as pltpu

N_DEV = 16
M, K_TOT, N = 4096, 4096, 8192
K_SH = K_TOT // N_DEV
TM = 512
TN = 1024


def kernel(x, w_mat, scale_x, scale_w):
    x8 = x.astype(jnp.float8_e4m3fn)
    w8 = w_mat.astype(jnp.float8_e4m3fn)

    def body(x_ref, w_ref, sx_ref, sw_ref, o_ref, wf_ref,
             xg, wstage, lx_sem, lw_sem, ls_sem, ps_sem,
             xs_send, xs_recv, ws_send, ws_recv,
             xl_send, xl_recv, wl_send, wl_recv):
        ni = pl.program_id(0)
        mi = pl.program_id(1)
        pid = ni * (M // TM) + mi
        my = lax.axis_index("i")
        right = lax.rem(my + 1, N_DEV)
        left = lax.rem(my + N_DEV - 1, N_DEV)

        @pl.when(pid == 0)
        def _gather():
            barrier = pltpu.get_barrier_semaphore()
            pl.semaphore_signal(barrier, inc=1, device_id=(left,),
                                device_id_type=pl.DeviceIdType.MESH)
            pl.semaphore_signal(barrier, inc=1, device_id=(right,),
                                device_id_type=pl.DeviceIdType.MESH)
            pl.semaphore_wait(barrier, 2)

            cx = pltpu.make_async_copy(
                x_ref, xg.at[:, pl.ds(my * K_SH, K_SH)], lx_sem)
            cw = pltpu.make_async_copy(
                w_ref, wf_ref.at[pl.ds(my * K_SH, K_SH), :], lw_sem)
            cx.start()
            cw.start()
            cx.wait()
            cw.wait()

            H_R, H_L = 8, 7
            for h in range(H_R):
                sr = lax.rem(my - h + N_DEV, N_DEV)
                xr = xg.at[:, pl.ds(sr * K_SH, K_SH)]
                wr = wf_ref.at[pl.ds(sr * K_SH, K_SH), :]
                rxr = pltpu.make_async_remote_copy(
                    src_ref=xr, dst_ref=xr,
                    send_sem=xs_send.at[h], recv_sem=xs_recv.at[h],
                    device_id=(right,), device_id_type=pl.DeviceIdType.MESH)
                rwr = pltpu.make_async_remote_copy(
                    src_ref=wr, dst_ref=wr,
                    send_sem=ws_send.at[h], recv_sem=ws_recv.at[h],
                    device_id=(right,), device_id_type=pl.DeviceIdType.MESH)
                rxr.start()
                rwr.start()
                if h < H_L:
                    sl = lax.rem(my + h, N_DEV)
                    xl = xg.at[:, pl.ds(sl * K_SH, K_SH)]
                    wl = wf_ref.at[pl.ds(sl * K_SH, K_SH), :]
                    rxl = pltpu.make_async_remote_copy(
                        src_ref=xl, dst_ref=xl,
                        send_sem=xl_send.at[h], recv_sem=xl_recv.at[h],
                        device_id=(left,),
                        device_id_type=pl.DeviceIdType.MESH)
                    rwl = pltpu.make_async_remote_copy(
                        src_ref=wl, dst_ref=wl,
                        send_sem=wl_send.at[h], recv_sem=wl_recv.at[h],
                        device_id=(left,),
                        device_id_type=pl.DeviceIdType.MESH)
                    rxl.start()
                    rwl.start()
                    rxl.wait()
                    rwl.wait()
                rxr.wait()
                rwr.wait()

        nb = lax.rem(ni, 2)

        @pl.when(mi == 0)
        def _stage_w():
            @pl.when(ni == 0)
            def _first():
                c0 = pltpu.make_async_copy(
                    wf_ref.at[:, pl.ds(0, TN)], wstage.at[0], ls_sem)
                c0.start()
                c0.wait()

            @pl.when(ni > 0)
            def _wait_prefetch():
                cp = pltpu.make_async_copy(
                    wf_ref.at[:, pl.ds(ni * TN, TN)], wstage.at[nb], ps_sem)
                cp.wait()

            @pl.when(ni < (N // TN) - 1)
            def _prefetch_next():
                cn = pltpu.make_async_copy(
                    wf_ref.at[:, pl.ds((ni + 1) * TN, TN)],
                    wstage.at[lax.rem(ni + 1, 2)], ps_sem)
                cn.start()

        m0 = mi * TM
        acc = jnp.dot(xg[pl.ds(m0, TM), :], wstage[nb],
                      preferred_element_type=jnp.float32)
        y = acc * (sx_ref[0] * sw_ref[0])
        yc = jnp.clip(y, -60.0, 60.0)
        o_ref[...] = y * (1.0 / (1.0 + jnp.exp(-yc)))

    out, _ = pl.pallas_call(
        body,
        grid=(N // TN, M // TM),
        out_shape=[
            jax.ShapeDtypeStruct((M, N), jnp.float32),
            jax.ShapeDtypeStruct((K_TOT, N), jnp.float8_e4m3fn),
        ],
        in_specs=[
            pl.BlockSpec(memory_space=pltpu.VMEM),
            pl.BlockSpec(memory_space=pltpu.VMEM),
            pl.BlockSpec(memory_space=pltpu.SMEM),
            pl.BlockSpec(memory_space=pltpu.SMEM),
        ],
        out_specs=[
            pl.BlockSpec((TM, TN), lambda n, m: (m, n)),
            pl.BlockSpec(memory_space=pl.ANY),
        ],
        scratch_shapes=[
            pltpu.VMEM((M, K_TOT), jnp.float8_e4m3fn),
            pltpu.VMEM((2, K_TOT, TN), jnp.float8_e4m3fn),
            pltpu.SemaphoreType.DMA,
            pltpu.SemaphoreType.DMA,
            pltpu.SemaphoreType.DMA,
            pltpu.SemaphoreType.DMA,
            pltpu.SemaphoreType.DMA((8,)),
            pltpu.SemaphoreType.DMA((8,)),
            pltpu.SemaphoreType.DMA((8,)),
            pltpu.SemaphoreType.DMA((8,)),
            pltpu.SemaphoreType.DMA((7,)),
            pltpu.SemaphoreType.DMA((7,)),
            pltpu.SemaphoreType.DMA((7,)),
            pltpu.SemaphoreType.DMA((7,)),
        ],
        compiler_params=pltpu.CompilerParams(
            collective_id=0,
            dimension_semantics=("arbitrary", "arbitrary"),
        ),
    )(x8, w8, scale_x, scale_w)
    return out
